# baseline (speedup 1.0000x reference)
import jax
import jax.numpy as jnp
from jax import lax
from jax.experimental import pallas as pl
from jax.experimental.pallas import tpu as pltpu
from jax.experimental.pallas import tpu_sc as plsc

_N_ATOMS = 819200
_N_FEATURES = 64
_TABLE_ROWS = 83
_NC = 2
_NS = 16
_NUM_WORKERS = _NC * _NS
_B_PER_W = _N_ATOMS // _NUM_WORKERS
_CHUNK = 320
_N_CHUNKS = _B_PER_W // _CHUNK
_L = 16
_NBLK = _N_FEATURES // _L


def _emb_body(x_hbm, table_hbm, out_hbm, win_sh, table_v, idx_v,
              rows0, rows1, ssem0, ssem1, dsem):
    cid = lax.axis_index("c")
    sid = lax.axis_index("s")
    wid = sid * _NC + cid
    base = wid * _B_PER_W

    pltpu.sync_copy(table_hbm, table_v)
    pltpu.sync_copy(x_hbm.at[pl.ds(base, _B_PER_W)], idx_v)

    rows = (rows0, rows1)
    ssem = (ssem0, ssem1)
    lanes = lax.iota(jnp.int32, _L)

    def expand(i, b):
        rbuf = rows[b]

        def group(g, carry):
            r16 = idx_v[pl.ds(i * _CHUNK + g * _L, _L)]
            p16 = lanes + g * _L
            for blk in range(_NBLK):
                for k in range(_L):
                    col = ((lanes + k) & (_L - 1)) + blk * _L
                    val = plsc.load_gather(table_v, [r16, col])
                    plsc.store_scatter(rbuf, [p16, col], val)
            return carry

        lax.fori_loop(0, _CHUNK // _L, group, 0)

    def fire_spill(b):
        pltpu.async_copy(rows[b], win_sh.at[b, sid], ssem[b])

    def wait_spill(b):
        pltpu.make_async_copy(rows[b], win_sh.at[b, sid], ssem[b]).wait()

    def fire_dma(i, v):
        for s2 in range(_NS):
            dst0 = (s2 * _NC + cid) * _B_PER_W + i * _CHUNK
            pltpu.async_copy(win_sh.at[v, s2],
                             out_hbm.at[pl.ds(dst0, _CHUNK)], dsem)

    def wait_dma(v):
        for s2 in range(_NS):
            pltpu.make_async_copy(win_sh.at[v, s2],
                                  out_hbm.at[pl.ds(0, _CHUNK)], dsem).wait()

    def step(i, b, first, expand_next):
        if not first:
            @pl.when(sid == 0)
            def _drain():
                wait_dma(b)
        plsc.subcore_barrier()
        fire_spill(b)
        if expand_next:
            expand(i + 1, 1 - b)
        wait_spill(b)
        plsc.subcore_barrier()

        @pl.when(sid == 0)
        def _push():
            fire_dma(i, b)

    expand(0, 0)
    step(0, 0, True, True)
    step(1, 1, True, True)

    def pair(j, carry):
        for b in range(2):
            step(2 * j + b, b, False, True)
        return carry

    lax.fori_loop(1, _N_CHUNKS // 2 - 1, pair, 0)

    step(_N_CHUNKS - 2, 0, False, True)
    step(_N_CHUNKS - 1, 1, False, False)

    @pl.when(sid == 0)
    def _drain_tail():
        wait_dma(0)
        wait_dma(1)

    plsc.subcore_barrier()


@jax.jit
def kernel(x, embedding_list):
    run = pl.kernel(
        _emb_body,
        out_type=jax.ShapeDtypeStruct((_N_ATOMS, _N_FEATURES), jnp.float32),
        mesh=plsc.VectorSubcoreMesh(core_axis_name="c", subcore_axis_name="s"),
        scratch_types=[
            pltpu.VMEM_SHARED((2, _NS, _CHUNK, _N_FEATURES), jnp.float32),
            pltpu.VMEM((_TABLE_ROWS, _N_FEATURES), jnp.float32),
            pltpu.VMEM((_B_PER_W,), jnp.int32),
            pltpu.VMEM((_CHUNK, _N_FEATURES), jnp.float32),
            pltpu.VMEM((_CHUNK, _N_FEATURES), jnp.float32),
            pltpu.SemaphoreType.DMA,
            pltpu.SemaphoreType.DMA,
            pltpu.SemaphoreType.DMA,
        ],
        compiler_params=pltpu.CompilerParams(use_tc_tiling_on_sc=False,
                                             needs_layout_passes=False),
    )
    return run(x, embedding_list)


# speedup vs baseline: 1.5230x; 1.5230x over previous
import jax
import jax.numpy as jnp
from jax import lax
from jax.experimental import pallas as pl
from jax.experimental.pallas import tpu as pltpu
from jax.experimental.pallas import tpu_sc as plsc

_N_ATOMS = 819200
_N_FEATURES = 64
_TABLE_ROWS = 83
_NC = 2
_NS = 16
_NUM_WORKERS = _NC * _NS
_B_PER_W = _N_ATOMS // _NUM_WORKERS
_CHUNK = 800
_N_CHUNKS = _B_PER_W // _CHUNK
_N_REP = 8


def _emb_body(x_hbm, table_hbm, out_hbm, table_sh, idx_v, rows0, rows1,
              semb0, semb1, semc0, semc1):
    cid = lax.axis_index("c")
    sid = lax.axis_index("s")
    wid = sid * _NC + cid
    base = wid * _B_PER_W

    rep = sid % _N_REP

    @pl.when(sid < _N_REP)
    def _stage_table():
        pltpu.sync_copy(table_hbm, table_sh.at[sid])

    pltpu.sync_copy(x_hbm.at[pl.ds(base, _B_PER_W)], idx_v)
    plsc.subcore_barrier()

    rows = (rows0, rows1)
    semb = (semb0, semb1)
    semc = (semc0, semc1)

    def gather(i, b):
        return pltpu.async_copy(
            table_sh.at[rep].at[idx_v.at[pl.ds(i * _CHUNK, _CHUNK)]],
            rows[b], semb[b])

    def put(i, b):
        return pltpu.async_copy(
            rows[b], out_hbm.at[pl.ds(base + i * _CHUNK, _CHUNK)], semc[b])

    g0 = gather(0, 0)
    g1 = gather(1, 1)
    g0.wait()
    put(0, 0)

    def pair(j, carry):
        i0 = 2 * j
        for b in range(2):
            i = i0 + b
            pltpu.make_async_copy(rows[b], out_hbm.at[pl.ds(0, _CHUNK)],
                                  semc[b]).wait()
            gather(i, b)
            pltpu.make_async_copy(
                table_sh.at[rep].at[idx_v.at[pl.ds(0, _CHUNK)]], rows[1 - b],
                semb[1 - b]).wait()
            put(i - 1, 1 - b)
        return carry

    lax.fori_loop(1, _N_CHUNKS // 2, pair, 0)

    pltpu.make_async_copy(
        table_sh.at[rep].at[idx_v.at[pl.ds(0, _CHUNK)]], rows1, semb1).wait()
    put(_N_CHUNKS - 1, 1)
    pltpu.make_async_copy(rows0, out_hbm.at[pl.ds(0, _CHUNK)], semc0).wait()
    pltpu.make_async_copy(rows1, out_hbm.at[pl.ds(0, _CHUNK)], semc1).wait()


@jax.jit
def kernel(x, embedding_list):
    run = pl.kernel(
        _emb_body,
        out_type=jax.ShapeDtypeStruct((_N_ATOMS, _N_FEATURES), jnp.float32),
        mesh=plsc.VectorSubcoreMesh(core_axis_name="c", subcore_axis_name="s"),
        scratch_types=[
            pltpu.VMEM_SHARED((_N_REP, _TABLE_ROWS, _N_FEATURES), jnp.float32),
            pltpu.VMEM((_B_PER_W,), jnp.int32),
            pltpu.VMEM((_CHUNK, _N_FEATURES), jnp.float32),
            pltpu.VMEM((_CHUNK, _N_FEATURES), jnp.float32),
            pltpu.SemaphoreType.DMA,
            pltpu.SemaphoreType.DMA,
            pltpu.SemaphoreType.DMA,
            pltpu.SemaphoreType.DMA,
        ],
        compiler_params=pltpu.CompilerParams(use_tc_tiling_on_sc=False),
    )
    return run(x, embedding_list)
